# baseline (speedup 1.0000x reference)
import functools
import math

import jax
import jax.numpy as jnp
from jax import lax
from jax.experimental import pallas as pl
from jax.experimental.pallas import tpu as pltpu
from jax.experimental.pallas import tpu_sc as plsc

_TOP_K_PERCENT = 0.25

_NW = 32
_NB = 4096
_L = 16


def _sc_topk_mean(x, B, C, H, W, k):
    rows = B * C
    rpw = rows // _NW
    im_rows = 96
    nchunks = H // im_rows
    nvr = W // _L
    kf = float(k)
    cpw = C // (_NW // B)

    mesh = plsc.VectorSubcoreMesh(core_axis_name="c", subcore_axis_name="s")

    @functools.partial(
        pl.kernel,
        out_type=[jax.ShapeDtypeStruct((_NW, 32), jnp.float32),
                  jax.ShapeDtypeStruct((_NW, 32), jnp.float32)],
        mesh=mesh,
        compiler_params=pltpu.CompilerParams(needs_layout_passes=False),
        scratch_types=[
            pltpu.VMEM((im_rows, W), jnp.float32),
            pltpu.VMEM((im_rows, W), jnp.float32),
            pltpu.VMEM((4 * _NB,), jnp.float32),
            pltpu.VMEM((4 * _NB,), jnp.float32),
            pltpu.VMEM((_NB,), jnp.float32),
            pltpu.VMEM((32,), jnp.float32),
            pltpu.VMEM((32,), jnp.float32),
            pltpu.SemaphoreType.DMA,
            pltpu.SemaphoreType.DMA,
        ],
    )
    def body(x_hbm, logit_hbm, prob_hbm, buf0, buf1, cnt1, sum1, cnt2,
             rlog, rprob, sem0, sem1):
        bufs = (buf0, buf1)
        sems = (sem0, sem1)
        wid = lax.axis_index("s") * 2 + lax.axis_index("c")
        bi = lax.shift_right_logical(wid, 3)
        ci0 = lax.bitwise_and(wid, jnp.int32(7)) * cpw
        iota = lax.iota(jnp.int32, _L)
        ones = jnp.ones((_L,), jnp.float32)
        zeros = jnp.zeros((_L,), jnp.float32)
        laneq = lax.shift_left(jnp.bitwise_and(iota, jnp.int32(3)), 12)

        def per_row(r, res):
            res0, res1 = res
            ci = ci0 + r

            @plsc.parallel_loop(0, _NB // 16, 1, unroll=2)
            def zbody(i):
                for u in range(4):
                    off = i * 64 + u * _L
                    cnt1[pl.ds(off, _L)] = zeros
                    sum1[pl.ds(off, _L)] = zeros

            @plsc.parallel_loop(0, _NB // 64, 1, unroll=2)
            def zbody2(i):
                for u in range(4):
                    off = i * 64 + u * _L
                    cnt2[pl.ds(off, _L)] = zeros

            def src(c):
                return x_hbm.at[bi, ci, pl.ds(c * im_rows, im_rows)]

            hdl = [pltpu.async_copy(src(0), buf0, sem0), None]
            for c in range(nchunks):
                if c + 1 < nchunks:
                    hdl[(c + 1) % 2] = pltpu.async_copy(
                        src(c + 1), bufs[(c + 1) % 2], sems[(c + 1) % 2])
                hdl[c % 2].wait()
                bc = bufs[c % 2]

                @plsc.parallel_loop(0, im_rows, 1, unroll=2)
                def v1(ir, bc=bc):
                    for u in range(nvr):
                        xv = bc[ir, pl.ds(u * _L, _L)]
                        raw = lax.bitcast_convert_type(xv, jnp.int32)
                        b = lax.shift_right_logical(raw, 20) + laneq
                        plsc.addupdate_scatter(cnt1, [b], ones)
                        plsc.addupdate_scatter(sum1, [b], xv)

            half = _NB // _L // 2
            def s1(t, carry):
                cum_c, cum_s, bstar, cnt_ab, sum_ab = carry
                posseg = t < half
                j = jnp.where(posseg, half - 1 - t, t)
                base = j * _L
                c = (cnt1[pl.ds(base, _L)] + cnt1[pl.ds(_NB + base, _L)]
                     + cnt1[pl.ds(2 * _NB + base, _L)]
                     + cnt1[pl.ds(3 * _NB + base, _L)])
                s = (sum1[pl.ds(base, _L)] + sum1[pl.ds(_NB + base, _L)]
                     + sum1[pl.ds(2 * _NB + base, _L)]
                     + sum1[pl.ds(3 * _NB + base, _L)])
                tc = jnp.sum(c)
                ts = jnp.sum(s)
                pc = lax.cumsum(c, axis=0)
                ps = lax.cumsum(s, axis=0)
                ac = cum_c + jnp.where(posseg, tc - pc, pc - c)
                asm = cum_s + jnp.where(posseg, ts - ps, ps - s)
                sel = jnp.logical_and(ac < kf, ac + c >= kf)
                hit = jnp.any(sel)
                lanes = j * _L + iota
                bstar = jnp.where(hit, jnp.max(jnp.where(sel, lanes, -1)),
                                  bstar)
                cnt_ab = jnp.where(hit, jnp.sum(jnp.where(sel, ac, 0.0)),
                                   cnt_ab)
                sum_ab = jnp.where(hit, jnp.sum(jnp.where(sel, asm, 0.0)),
                                   sum_ab)
                return (cum_c + tc, cum_s + ts, bstar, cnt_ab, sum_ab)
            _, _, bstar, cnt_ab, sum_ab = lax.fori_loop(
                0, _NB // _L, s1, (0.0, 0.0, jnp.int32(0), 0.0, 0.0))

            hdl = [pltpu.async_copy(src(0), buf0, sem0), None]
            for c in range(nchunks):
                if c + 1 < nchunks:
                    hdl[(c + 1) % 2] = pltpu.async_copy(
                        src(c + 1), bufs[(c + 1) % 2], sems[(c + 1) % 2])
                hdl[c % 2].wait()
                bc = bufs[c % 2]

                @plsc.parallel_loop(0, im_rows, 1, unroll=2)
                def v2(ir, bc=bc):
                    for u in range(nvr):
                        xv = bc[ir, pl.ds(u * _L, _L)]
                        raw = lax.bitcast_convert_type(xv, jnp.int32)
                        b = lax.shift_right_logical(raw, 20)
                        msk = b == bstar
                        fine = jnp.bitwise_and(
                            lax.shift_right_logical(raw, 8), jnp.int32(0xFFF))
                        plsc.addupdate_scatter(cnt2, [fine], ones, mask=msk)

            kk = kf - cnt_ab
            base_key = lax.shift_left(bstar, 20)
            posb = bstar < jnp.int32(_NB // 2)

            def s2(t, carry):
                cum_c, cum_s, f_ab, f_abs, bmean = carry
                j = jnp.where(posb, _NB // _L - 1 - t, t)
                c = cnt2[pl.ds(j * _L, _L)]
                lanes = j * _L + iota
                keymid = base_key + lax.shift_left(lanes, 8) + 128
                val = lax.bitcast_convert_type(keymid, jnp.float32)
                s = c * val
                tc = jnp.sum(c)
                ts = jnp.sum(s)
                pc = lax.cumsum(c, axis=0)
                ps = lax.cumsum(s, axis=0)
                ac = cum_c + jnp.where(posb, tc - pc, pc - c)
                asm = cum_s + jnp.where(posb, ts - ps, ps - s)
                sel = jnp.logical_and(ac < kk, ac + c >= kk)
                hit = jnp.any(sel)
                f_ab = jnp.where(hit, jnp.sum(jnp.where(sel, ac, 0.0)), f_ab)
                f_abs = jnp.where(hit, jnp.sum(jnp.where(sel, asm, 0.0)),
                                  f_abs)
                bmean = jnp.where(hit, jnp.sum(jnp.where(sel, val, 0.0)),
                                  bmean)
                return (cum_c + tc, cum_s + ts, f_ab, f_abs, bmean)
            _, _, f_ab, f_abs, bmean = lax.fori_loop(
                0, _NB // _L, s2, (0.0, 0.0, 0.0, 0.0, 0.0))

            rem = kk - f_ab
            logit_v = jnp.full((_L,),
                               (sum_ab + f_abs + rem * bmean) * (1.0 / kf))

            res0 = jnp.where(iota == r, logit_v, res0)
            res1 = jnp.where(iota == r - _L, logit_v, res1)
            return (res0, res1)

        res0, res1 = lax.fori_loop(0, rpw, per_row, (zeros, zeros))
        p0 = 1.0 / (1.0 + jnp.exp(-res0))
        p1 = 1.0 / (1.0 + jnp.exp(-res1))
        rlog[pl.ds(0, _L)] = res0
        rlog[pl.ds(_L, _L)] = res1
        rprob[pl.ds(0, _L)] = p0
        rprob[pl.ds(_L, _L)] = p1
        pltpu.sync_copy(rlog, logit_hbm.at[wid])
        pltpu.sync_copy(rprob, prob_hbm.at[wid])

    return body(x)


def kernel(mask_logits):
    B, C, H, W = mask_logits.shape
    rows = B * C
    n_pixels = H * W
    k = math.ceil(n_pixels * _TOP_K_PERCENT)
    lg, pb = _sc_topk_mean(mask_logits, B, C, H, W, k)
    rpw = rows // _NW
    logits = lg[:, :rpw].reshape(B, C)
    probs = pb[:, :rpw].reshape(B, C)
    return (logits, probs)


# speedup vs baseline: 1.1101x; 1.1101x over previous
import functools
import math

import jax
import jax.numpy as jnp
from jax import lax
from jax.experimental import pallas as pl
from jax.experimental.pallas import tpu as pltpu
from jax.experimental.pallas import tpu_sc as plsc

_TOP_K_PERCENT = 0.25

_NW = 32
_NB = 4096
_L = 16


def _sc_topk_mean(x, B, C, H, W, k):
    rows = B * C
    rpw = rows // _NW
    im_rows = 96
    nchunks = H // im_rows
    nvr = W // _L
    kf = float(k)
    cpw = C // (_NW // B)

    mesh = plsc.VectorSubcoreMesh(core_axis_name="c", subcore_axis_name="s")

    @functools.partial(
        pl.kernel,
        out_type=[jax.ShapeDtypeStruct((_NW, 32), jnp.float32),
                  jax.ShapeDtypeStruct((_NW, 32), jnp.float32)],
        mesh=mesh,
        compiler_params=pltpu.CompilerParams(needs_layout_passes=False),
        scratch_types=[
            pltpu.VMEM((im_rows, W), jnp.float32),
            pltpu.VMEM((im_rows, W), jnp.float32),
            pltpu.VMEM((_NB,), jnp.float32),
            pltpu.VMEM((_NB,), jnp.float32),
            pltpu.VMEM((_L,), jnp.float32),
            pltpu.VMEM((32,), jnp.float32),
            pltpu.VMEM((32,), jnp.float32),
            pltpu.SemaphoreType.DMA,
            pltpu.SemaphoreType.DMA,
        ],
    )
    def body(x_hbm, logit_hbm, prob_hbm, buf0, buf1, cnt1, cnt2, acc,
             rlog, rprob, sem0, sem1):
        bufs = (buf0, buf1)
        sems = (sem0, sem1)
        wid = lax.axis_index("s") * 2 + lax.axis_index("c")
        bi = lax.shift_right_logical(wid, 3)
        ci0 = lax.bitwise_and(wid, jnp.int32(7)) * cpw
        iota = lax.iota(jnp.int32, _L)
        ones = jnp.ones((_L,), jnp.float32)
        zeros = jnp.zeros((_L,), jnp.float32)

        def per_row(r, res):
            res0, res1 = res
            ci = ci0 + r

            @plsc.parallel_loop(0, _NB // 64, 1, unroll=2)
            def zbody(i):
                for u in range(4):
                    off = i * 64 + u * _L
                    cnt1[pl.ds(off, _L)] = zeros
                    cnt2[pl.ds(off, _L)] = zeros
            acc[pl.ds(0, _L)] = zeros

            def src(c):
                return x_hbm.at[bi, ci, pl.ds(c * im_rows, im_rows)]

            hdl = [pltpu.async_copy(src(0), buf0, sem0), None]
            for c in range(nchunks):
                if c + 1 < nchunks:
                    hdl[(c + 1) % 2] = pltpu.async_copy(
                        src(c + 1), bufs[(c + 1) % 2], sems[(c + 1) % 2])
                hdl[c % 2].wait()
                bc = bufs[c % 2]

                @plsc.parallel_loop(0, im_rows, 1, unroll=2)
                def v1(ir, bc=bc):
                    for u in range(nvr):
                        xv = bc[ir, pl.ds(u * _L, _L)]
                        raw = lax.bitcast_convert_type(xv, jnp.int32)
                        b = lax.shift_right_logical(raw, 20)
                        plsc.addupdate_scatter(cnt1, [b], ones)

            half = _NB // _L // 2
            def s1(t, carry):
                cum_c, bstar, cnt_ab = carry
                posseg = t < half
                j = jnp.where(posseg, half - 1 - t, t)
                c = cnt1[pl.ds(j * _L, _L)]
                tc = jnp.sum(c)
                pc = lax.cumsum(c, axis=0)
                ac = cum_c + jnp.where(posseg, tc - pc, pc - c)
                sel = jnp.logical_and(ac < kf, ac + c >= kf)
                hit = jnp.any(sel)
                lanes = j * _L + iota
                bstar = jnp.where(hit, jnp.max(jnp.where(sel, lanes, -1)),
                                  bstar)
                cnt_ab = jnp.where(hit, jnp.sum(jnp.where(sel, ac, 0.0)),
                                   cnt_ab)
                return (cum_c + tc, bstar, cnt_ab)
            _, bstar, cnt_ab = lax.fori_loop(
                0, _NB // _L, s1, (0.0, jnp.int32(0), 0.0))
            posb = bstar < jnp.int32(_NB // 2)

            hdl = [pltpu.async_copy(src(0), buf0, sem0), None]
            for c in range(nchunks):
                if c + 1 < nchunks:
                    hdl[(c + 1) % 2] = pltpu.async_copy(
                        src(c + 1), bufs[(c + 1) % 2], sems[(c + 1) % 2])
                hdl[c % 2].wait()
                bc = bufs[c % 2]

                @plsc.parallel_loop(0, im_rows, 1, unroll=2)
                def v2(ir, bc=bc):
                    part = zeros
                    for u in range(nvr):
                        xv = bc[ir, pl.ds(u * _L, _L)]
                        raw = lax.bitcast_convert_type(xv, jnp.int32)
                        b = lax.shift_right_logical(raw, 20)
                        msk = b == bstar
                        above = jnp.where(
                            posb,
                            jnp.logical_and(b > bstar, b < jnp.int32(_NB // 2)),
                            b < bstar)
                        part = part + jnp.where(above, xv, 0.0)
                        fine = jnp.bitwise_and(
                            lax.shift_right_logical(raw, 8), jnp.int32(0xFFF))
                        plsc.addupdate_scatter(cnt2, [fine], ones, mask=msk)
                    plsc.addupdate_scatter(acc, [iota], part)
            sum_ab = jnp.sum(acc[pl.ds(0, _L)])

            kk = kf - cnt_ab
            base_key = lax.shift_left(bstar, 20)

            def s2(t, carry):
                cum_c, cum_s, f_ab, f_abs, bmean = carry
                j = jnp.where(posb, _NB // _L - 1 - t, t)
                c = cnt2[pl.ds(j * _L, _L)]
                lanes = j * _L + iota
                keymid = base_key + lax.shift_left(lanes, 8) + 128
                val = lax.bitcast_convert_type(keymid, jnp.float32)
                s = c * val
                tc = jnp.sum(c)
                ts = jnp.sum(s)
                pc = lax.cumsum(c, axis=0)
                ps = lax.cumsum(s, axis=0)
                ac = cum_c + jnp.where(posb, tc - pc, pc - c)
                asm = cum_s + jnp.where(posb, ts - ps, ps - s)
                sel = jnp.logical_and(ac < kk, ac + c >= kk)
                hit = jnp.any(sel)
                f_ab = jnp.where(hit, jnp.sum(jnp.where(sel, ac, 0.0)), f_ab)
                f_abs = jnp.where(hit, jnp.sum(jnp.where(sel, asm, 0.0)),
                                  f_abs)
                bmean = jnp.where(hit, jnp.sum(jnp.where(sel, val, 0.0)),
                                  bmean)
                return (cum_c + tc, cum_s + ts, f_ab, f_abs, bmean)
            _, _, f_ab, f_abs, bmean = lax.fori_loop(
                0, _NB // _L, s2, (0.0, 0.0, 0.0, 0.0, 0.0))

            rem = kk - f_ab
            logit_v = jnp.full((_L,),
                               (sum_ab + f_abs + rem * bmean) * (1.0 / kf))

            res0 = jnp.where(iota == r, logit_v, res0)
            res1 = jnp.where(iota == r - _L, logit_v, res1)
            return (res0, res1)

        res0, res1 = lax.fori_loop(0, rpw, per_row, (zeros, zeros))
        p0 = 1.0 / (1.0 + jnp.exp(-res0))
        p1 = 1.0 / (1.0 + jnp.exp(-res1))
        rlog[pl.ds(0, _L)] = res0
        rlog[pl.ds(_L, _L)] = res1
        rprob[pl.ds(0, _L)] = p0
        rprob[pl.ds(_L, _L)] = p1
        pltpu.sync_copy(rlog, logit_hbm.at[wid])
        pltpu.sync_copy(rprob, prob_hbm.at[wid])

    return body(x)


def kernel(mask_logits):
    B, C, H, W = mask_logits.shape
    rows = B * C
    n_pixels = H * W
    k = math.ceil(n_pixels * _TOP_K_PERCENT)
    lg, pb = _sc_topk_mean(mask_logits, B, C, H, W, k)
    rpw = rows // _NW
    logits = lg[:, :rpw].reshape(B, C)
    probs = pb[:, :rpw].reshape(B, C)
    return (logits, probs)


# speedup vs baseline: 1.1111x; 1.0009x over previous
import functools
import math

import jax
import jax.numpy as jnp
from jax import lax
from jax.experimental import pallas as pl
from jax.experimental.pallas import tpu as pltpu
from jax.experimental.pallas import tpu_sc as plsc

_TOP_K_PERCENT = 0.25

_NW = 32
_NB = 4096
_L = 16


def _sc_topk_mean(x, B, C, H, W, k):
    rows = B * C
    rpw = rows // _NW
    im_rows = 96
    nchunks = H // im_rows
    nvr = W // _L
    kf = float(k)
    cpw = C // (_NW // B)

    mesh = plsc.VectorSubcoreMesh(core_axis_name="c", subcore_axis_name="s")

    @functools.partial(
        pl.kernel,
        out_type=[jax.ShapeDtypeStruct((_NW, 32), jnp.float32),
                  jax.ShapeDtypeStruct((_NW, 32), jnp.float32)],
        mesh=mesh,
        compiler_params=pltpu.CompilerParams(needs_layout_passes=False),
        scratch_types=[
            pltpu.VMEM((im_rows, W), jnp.float32),
            pltpu.VMEM((im_rows, W), jnp.float32),
            pltpu.VMEM((_NB,), jnp.float32),
            pltpu.VMEM((_NB,), jnp.float32),
            pltpu.VMEM((_L,), jnp.float32),
            pltpu.VMEM((32,), jnp.float32),
            pltpu.VMEM((32,), jnp.float32),
            pltpu.SemaphoreType.DMA,
            pltpu.SemaphoreType.DMA,
        ],
    )
    def body(x_hbm, logit_hbm, prob_hbm, buf0, buf1, cnt1, cnt2, acc,
             rlog, rprob, sem0, sem1):
        bufs = (buf0, buf1)
        sems = (sem0, sem1)
        wid = lax.axis_index("s") * 2 + lax.axis_index("c")
        bi = lax.shift_right_logical(wid, 3)
        ci0 = lax.bitwise_and(wid, jnp.int32(7)) * cpw
        iota = lax.iota(jnp.int32, _L)
        ones = jnp.ones((_L,), jnp.float32)
        zeros = jnp.zeros((_L,), jnp.float32)

        def per_row(r, res):
            res0, res1 = res
            ci = ci0 + r

            @plsc.parallel_loop(0, _NB // 64, 1, unroll=2)
            def zbody(i):
                for u in range(4):
                    off = i * 64 + u * _L
                    cnt1[pl.ds(off, _L)] = zeros
                    cnt2[pl.ds(off, _L)] = zeros
            acc[pl.ds(0, _L)] = zeros

            def src(c):
                return x_hbm.at[bi, ci, pl.ds(c * im_rows, im_rows)]

            hdl = [pltpu.async_copy(src(0), buf0, sem0), None]
            for c in range(nchunks):
                if c + 1 < nchunks:
                    hdl[(c + 1) % 2] = pltpu.async_copy(
                        src(c + 1), bufs[(c + 1) % 2], sems[(c + 1) % 2])
                hdl[c % 2].wait()
                bc = bufs[c % 2]

                @plsc.parallel_loop(0, im_rows, 1, unroll=1)
                def v1(ir, bc=bc):
                    for u in range(nvr):
                        xv = bc[ir, pl.ds(u * _L, _L)]
                        raw = lax.bitcast_convert_type(xv, jnp.int32)
                        b = lax.shift_right_logical(raw, 20)
                        plsc.addupdate_scatter(cnt1, [b], ones)

            half = _NB // _L // 2
            def s1(t, carry):
                cum_c, bstar, cnt_ab = carry
                posseg = t < half
                j = jnp.where(posseg, half - 1 - t, t)
                c = cnt1[pl.ds(j * _L, _L)]
                tc = jnp.sum(c)
                pc = lax.cumsum(c, axis=0)
                ac = cum_c + jnp.where(posseg, tc - pc, pc - c)
                sel = jnp.logical_and(ac < kf, ac + c >= kf)
                hit = jnp.any(sel)
                lanes = j * _L + iota
                bstar = jnp.where(hit, jnp.max(jnp.where(sel, lanes, -1)),
                                  bstar)
                cnt_ab = jnp.where(hit, jnp.sum(jnp.where(sel, ac, 0.0)),
                                   cnt_ab)
                return (cum_c + tc, bstar, cnt_ab)
            _, bstar, cnt_ab = lax.fori_loop(
                0, _NB // _L, s1, (0.0, jnp.int32(0), 0.0))
            posb = bstar < jnp.int32(_NB // 2)

            hdl = [pltpu.async_copy(src(0), buf0, sem0), None]
            for c in range(nchunks):
                if c + 1 < nchunks:
                    hdl[(c + 1) % 2] = pltpu.async_copy(
                        src(c + 1), bufs[(c + 1) % 2], sems[(c + 1) % 2])
                hdl[c % 2].wait()
                bc = bufs[c % 2]

                @plsc.parallel_loop(0, im_rows, 1, unroll=1)
                def v2(ir, bc=bc):
                    part = zeros
                    for u in range(nvr):
                        xv = bc[ir, pl.ds(u * _L, _L)]
                        raw = lax.bitcast_convert_type(xv, jnp.int32)
                        b = lax.shift_right_logical(raw, 20)
                        msk = b == bstar
                        above = jnp.where(
                            posb,
                            jnp.logical_and(b > bstar, b < jnp.int32(_NB // 2)),
                            b < bstar)
                        part = part + jnp.where(above, xv, 0.0)
                        fine = jnp.bitwise_and(
                            lax.shift_right_logical(raw, 8), jnp.int32(0xFFF))
                        plsc.addupdate_scatter(cnt2, [fine], ones, mask=msk)
                    plsc.addupdate_scatter(acc, [iota], part)
            sum_ab = jnp.sum(acc[pl.ds(0, _L)])

            kk = kf - cnt_ab
            base_key = lax.shift_left(bstar, 20)

            def s2(t, carry):
                cum_c, cum_s, f_ab, f_abs, bmean = carry
                j = jnp.where(posb, _NB // _L - 1 - t, t)
                c = cnt2[pl.ds(j * _L, _L)]
                lanes = j * _L + iota
                keymid = base_key + lax.shift_left(lanes, 8) + 128
                val = lax.bitcast_convert_type(keymid, jnp.float32)
                s = c * val
                tc = jnp.sum(c)
                ts = jnp.sum(s)
                pc = lax.cumsum(c, axis=0)
                ps = lax.cumsum(s, axis=0)
                ac = cum_c + jnp.where(posb, tc - pc, pc - c)
                asm = cum_s + jnp.where(posb, ts - ps, ps - s)
                sel = jnp.logical_and(ac < kk, ac + c >= kk)
                hit = jnp.any(sel)
                f_ab = jnp.where(hit, jnp.sum(jnp.where(sel, ac, 0.0)), f_ab)
                f_abs = jnp.where(hit, jnp.sum(jnp.where(sel, asm, 0.0)),
                                  f_abs)
                bmean = jnp.where(hit, jnp.sum(jnp.where(sel, val, 0.0)),
                                  bmean)
                return (cum_c + tc, cum_s + ts, f_ab, f_abs, bmean)
            _, _, f_ab, f_abs, bmean = lax.fori_loop(
                0, _NB // _L, s2, (0.0, 0.0, 0.0, 0.0, 0.0))

            rem = kk - f_ab
            logit_v = jnp.full((_L,),
                               (sum_ab + f_abs + rem * bmean) * (1.0 / kf))

            res0 = jnp.where(iota == r, logit_v, res0)
            res1 = jnp.where(iota == r - _L, logit_v, res1)
            return (res0, res1)

        res0, res1 = lax.fori_loop(0, rpw, per_row, (zeros, zeros))
        p0 = 1.0 / (1.0 + jnp.exp(-res0))
        p1 = 1.0 / (1.0 + jnp.exp(-res1))
        rlog[pl.ds(0, _L)] = res0
        rlog[pl.ds(_L, _L)] = res1
        rprob[pl.ds(0, _L)] = p0
        rprob[pl.ds(_L, _L)] = p1
        pltpu.sync_copy(rlog, logit_hbm.at[wid])
        pltpu.sync_copy(rprob, prob_hbm.at[wid])

    return body(x)


def kernel(mask_logits):
    B, C, H, W = mask_logits.shape
    rows = B * C
    n_pixels = H * W
    k = math.ceil(n_pixels * _TOP_K_PERCENT)
    lg, pb = _sc_topk_mean(mask_logits, B, C, H, W, k)
    rpw = rows // _NW
    logits = lg[:, :rpw].reshape(B, C)
    probs = pb[:, :rpw].reshape(B, C)
    return (logits, probs)


# speedup vs baseline: 1.3140x; 1.1826x over previous
import functools
import math

import jax
import jax.numpy as jnp
from jax import lax
from jax.experimental import pallas as pl
from jax.experimental.pallas import tpu as pltpu
from jax.experimental.pallas import tpu_sc as plsc

_TOP_K_PERCENT = 0.25

_NW = 32
_NB = 4096
_L = 16


def _sc_topk_mean(x, B, C, H, W, k):
    rows = B * C
    rpw = rows // _NW
    im_rows = 96
    nchunks = H // im_rows
    nvr = W // _L
    kf = float(k)
    cpw = C // (_NW // B)

    mesh = plsc.VectorSubcoreMesh(core_axis_name="c", subcore_axis_name="s")

    @functools.partial(
        pl.kernel,
        out_type=[jax.ShapeDtypeStruct((_NW, 32), jnp.float32),
                  jax.ShapeDtypeStruct((_NW, 32), jnp.float32)],
        mesh=mesh,
        compiler_params=pltpu.CompilerParams(needs_layout_passes=False),
        scratch_types=[
            pltpu.VMEM((im_rows, W), jnp.float32),
            pltpu.VMEM((im_rows, W), jnp.float32),
            pltpu.VMEM((_NB,), jnp.float32),
            pltpu.VMEM((_NB,), jnp.float32),
            pltpu.VMEM((_L,), jnp.float32),
            pltpu.VMEM((32,), jnp.float32),
            pltpu.VMEM((32,), jnp.float32),
            pltpu.SemaphoreType.DMA,
            pltpu.SemaphoreType.DMA,
        ],
    )
    def body(x_hbm, logit_hbm, prob_hbm, buf0, buf1, cnt1, cnt2, acc,
             rlog, rprob, sem0, sem1):
        bufs = (buf0, buf1)
        sems = (sem0, sem1)
        wid = lax.axis_index("s") * 2 + lax.axis_index("c")
        bi = lax.shift_right_logical(wid, 3)
        ci0 = lax.bitwise_and(wid, jnp.int32(7)) * cpw
        iota = lax.iota(jnp.int32, _L)
        ones = jnp.ones((_L,), jnp.float32)
        zeros = jnp.zeros((_L,), jnp.float32)

        def chunk_src(ci, c):
            return x_hbm.at[bi, ci, pl.ds(c * im_rows, im_rows)]

        pltpu.async_copy(chunk_src(ci0, 0), buf0, sem0)

        def per_row(r, res):
            res0, res1 = res
            ci = ci0 + r
            ci_next = ci0 + jnp.minimum(r + 1, rpw - 1)

            @plsc.parallel_loop(0, _NB // 64, 1, unroll=2)
            def zbody(i):
                for u in range(4):
                    off = i * 64 + u * _L
                    cnt1[pl.ds(off, _L)] = zeros
                    cnt2[pl.ds(off, _L)] = zeros
            acc[pl.ds(0, _L)] = zeros

            def src(c):
                return chunk_src(ci, c)

            def wait_chunk(c):
                pltpu.make_async_copy(src(c), bufs[c % 2],
                                      sems[c % 2]).wait()

            for c in range(nchunks):
                if c + 1 < nchunks:
                    pltpu.async_copy(
                        src(c + 1), bufs[(c + 1) % 2], sems[(c + 1) % 2])
                else:
                    pltpu.async_copy(src(0), buf0, sem0)
                wait_chunk(c)
                bc = bufs[c % 2]

                @plsc.parallel_loop(0, im_rows, 1, unroll=1)
                def v1(ir, bc=bc):
                    for u in range(nvr):
                        xv = bc[ir, pl.ds(u * _L, _L)]
                        raw = lax.bitcast_convert_type(xv, jnp.int32)
                        b = lax.shift_right_logical(raw, 20)
                        plsc.addupdate_scatter(cnt1, [b], ones)

            half = _NB // _L // 2
            def s1(t, carry):
                cum_c, bstar, cnt_ab = carry
                posseg = t < half
                j = jnp.where(posseg, half - 1 - t, t)
                c = cnt1[pl.ds(j * _L, _L)]
                tc = jnp.sum(c)
                pc = lax.cumsum(c, axis=0)
                ac = cum_c + jnp.where(posseg, tc - pc, pc - c)
                sel = jnp.logical_and(ac < kf, ac + c >= kf)
                hit = jnp.any(sel)
                lanes = j * _L + iota
                bstar = jnp.where(hit, jnp.max(jnp.where(sel, lanes, -1)),
                                  bstar)
                cnt_ab = jnp.where(hit, jnp.sum(jnp.where(sel, ac, 0.0)),
                                   cnt_ab)
                return (cum_c + tc, bstar, cnt_ab)
            _, bstar, cnt_ab = lax.fori_loop(
                0, _NB // _L, s1, (0.0, jnp.int32(0), 0.0))
            posb = bstar < jnp.int32(_NB // 2)
            lo = jnp.where(posb, bstar, jnp.int32(-1))
            hi = jnp.where(posb, jnp.int32(_NB // 2), bstar)

            for c in range(nchunks):
                if c + 1 < nchunks:
                    pltpu.async_copy(
                        src(c + 1), bufs[(c + 1) % 2], sems[(c + 1) % 2])
                else:
                    pltpu.async_copy(chunk_src(ci_next, 0), buf0, sem0)
                wait_chunk(c)
                bc = bufs[c % 2]

                @plsc.parallel_loop(0, im_rows, 1, unroll=1)
                def v2(ir, bc=bc):
                    part = zeros
                    for u in range(nvr):
                        xv = bc[ir, pl.ds(u * _L, _L)]
                        raw = lax.bitcast_convert_type(xv, jnp.int32)
                        b = lax.shift_right_logical(raw, 20)
                        msk = b == bstar
                        above = jnp.logical_and(b > lo, b < hi)
                        part = part + jnp.where(above, xv, 0.0)
                        fine = jnp.bitwise_and(
                            lax.shift_right_logical(raw, 8), jnp.int32(0xFFF))
                        plsc.addupdate_scatter(cnt2, [fine], ones, mask=msk)
                    plsc.addupdate_scatter(acc, [iota], part)
            sum_ab = jnp.sum(acc[pl.ds(0, _L)])

            kk = kf - cnt_ab
            base_key = lax.shift_left(bstar, 20)

            def s2(t, carry):
                cum_c, cum_s, f_ab, f_abs, bmean = carry
                j = jnp.where(posb, _NB // _L - 1 - t, t)
                c = cnt2[pl.ds(j * _L, _L)]
                lanes = j * _L + iota
                keymid = base_key + lax.shift_left(lanes, 8) + 128
                val = lax.bitcast_convert_type(keymid, jnp.float32)
                s = c * val
                tc = jnp.sum(c)
                ts = jnp.sum(s)
                pc = lax.cumsum(c, axis=0)
                ps = lax.cumsum(s, axis=0)
                ac = cum_c + jnp.where(posb, tc - pc, pc - c)
                asm = cum_s + jnp.where(posb, ts - ps, ps - s)
                sel = jnp.logical_and(ac < kk, ac + c >= kk)
                hit = jnp.any(sel)
                f_ab = jnp.where(hit, jnp.sum(jnp.where(sel, ac, 0.0)), f_ab)
                f_abs = jnp.where(hit, jnp.sum(jnp.where(sel, asm, 0.0)),
                                  f_abs)
                bmean = jnp.where(hit, jnp.sum(jnp.where(sel, val, 0.0)),
                                  bmean)
                return (cum_c + tc, cum_s + ts, f_ab, f_abs, bmean)
            _, _, f_ab, f_abs, bmean = lax.fori_loop(
                0, _NB // _L, s2, (0.0, 0.0, 0.0, 0.0, 0.0))

            rem = kk - f_ab
            logit_v = jnp.full((_L,),
                               (sum_ab + f_abs + rem * bmean) * (1.0 / kf))

            res0 = jnp.where(iota == r, logit_v, res0)
            res1 = jnp.where(iota == r - _L, logit_v, res1)
            return (res0, res1)

        res0, res1 = lax.fori_loop(0, rpw, per_row, (zeros, zeros))
        pltpu.make_async_copy(chunk_src(ci0 + rpw - 1, 0), buf0, sem0).wait()
        p0 = 1.0 / (1.0 + jnp.exp(-res0))
        p1 = 1.0 / (1.0 + jnp.exp(-res1))
        rlog[pl.ds(0, _L)] = res0
        rlog[pl.ds(_L, _L)] = res1
        rprob[pl.ds(0, _L)] = p0
        rprob[pl.ds(_L, _L)] = p1
        pltpu.sync_copy(rlog, logit_hbm.at[wid])
        pltpu.sync_copy(rprob, prob_hbm.at[wid])

    return body(x)


def kernel(mask_logits):
    B, C, H, W = mask_logits.shape
    rows = B * C
    n_pixels = H * W
    k = math.ceil(n_pixels * _TOP_K_PERCENT)
    lg, pb = _sc_topk_mean(mask_logits, B, C, H, W, k)
    rpw = rows // _NW
    logits = lg[:, :rpw].reshape(B, C)
    probs = pb[:, :rpw].reshape(B, C)
    return (logits, probs)
